# baseline (speedup 1.0000x reference)
import functools

import jax
import jax.numpy as jnp
from jax import lax
from jax.experimental import pallas as pl
from jax.experimental.pallas import tpu as pltpu
from jax.experimental.pallas import tpu_sc as plsc

N = 10000
NP = 10240
RB = 256
NBLK = NP // RB
E = 160000
EROWS = 1280
EP = EROWS * 128
RPT = EROWS // 16
SL = NP // 16
IB = 40

_MESH = plsc.VectorSubcoreMesh(core_axis_name="c", subcore_axis_name="s")
NEG_INF = float("-inf")


EPT = EP // 16


def _deg_body(srcf, dstf, zeros_h, out0, out1,
              idx_v, acc_v, red_v, res_v, stage_sh):
    c = lax.axis_index("c")
    s = lax.axis_index("s")
    ones16 = jnp.ones((16,), jnp.float32)
    pltpu.sync_copy(zeros_h, acc_v)
    for core, eref, out in ((0, srcf, out0), (1, dstf, out1)):
        @pl.when(c == core)
        def _(eref=eref, out=out):
            pltpu.sync_copy(eref.at[pl.ds(s * EPT, EPT)], idx_v)

            def step(b, carry):
                for k in range(8):
                    idx = idx_v[pl.ds(b * 128 + k * 16, 16)]
                    plsc.addupdate_scatter(acc_v, [idx], ones16)
                return carry

            lax.fori_loop(0, EPT // 128, step, 0)
            pltpu.sync_copy(acc_v, stage_sh.at[s])
            plsc.subcore_barrier()
            for t in range(16):
                pltpu.sync_copy(stage_sh.at[t].at[pl.ds(s * SL, SL)],
                                red_v.at[pl.ds(t * SL, SL)])

            def red(i, carry):
                tot = red_v[pl.ds(i * 16, 16)]
                for t in range(1, 16):
                    tot = tot + red_v[pl.ds(t * SL + i * 16, 16)]
                res_v[pl.ds(i * 16, 16)] = tot
                return carry

            lax.fori_loop(0, SL // 16, red, 0)
            pltpu.sync_copy(res_v, out.at[pl.ds(s * SL, SL)])

    return None


def _sc_degrees(srcf, dstf):
    zeros = jnp.zeros((NP,), jnp.float32)
    return pl.kernel(
        _deg_body,
        out_type=(jax.ShapeDtypeStruct((NP,), jnp.float32),
                  jax.ShapeDtypeStruct((NP,), jnp.float32)),
        mesh=_MESH,
        scratch_types=[
            pltpu.VMEM((EPT,), jnp.int32),
            pltpu.VMEM((NP,), jnp.float32),
            pltpu.VMEM((16 * SL,), jnp.float32),
            pltpu.VMEM((SL,), jnp.float32),
            pltpu.VMEM_SHARED((16, NP), jnp.float32),
        ],
        compiler_params=pltpu.CompilerParams(needs_layout_passes=False),
    )(srcf, dstf, zeros)


def _agg_body(*refs, fc, tasks, n_tables, n_slots):
    tables = refs[:n_tables]
    srcr, dstr, zeros_h = refs[n_tables:n_tables + 3]
    outs = refs[n_tables + 3:n_tables + 3 + n_slots]
    src_v, dst_v, rows0, rows1, acc_sh, sem0, sem1, ssem0, ssem1 = \
        refs[n_tables + 3 + n_slots:]
    c = lax.axis_index("c")
    s = lax.axis_index("s")
    for core in (0, 1):
        @pl.when(c == core)
        def _(core=core):
            for ci, row_lo, nrt, slot in tasks[core]:
                def gat(row, buf, sem, ci=ci):
                    return pltpu.make_async_copy(
                        tables[ci].at[src_v.at[row]], buf, sem)

                pltpu.sync_copy(zeros_h, acc_sh.at[pl.ds(s * SL, SL)])
                plsc.subcore_barrier()
                def sca(row, buf, sem):
                    return pltpu.make_async_copy(
                        buf, acc_sh.at[dst_v.at[row]], sem)

                for g in range(nrt // IB):
                    base = row_lo + s * nrt + g * IB
                    pltpu.sync_copy(srcr.at[pl.ds(base, IB)], src_v)
                    pltpu.sync_copy(dstr.at[pl.ds(base, IB)], dst_v)
                    gat(0, rows0, sem0).start()
                    gat(1, rows1, sem1).start()

                    def step(h, carry):
                        r = 2 * h
                        gat(r, rows0, sem0).wait()
                        pltpu.async_copy(rows0, acc_sh.at[dst_v.at[r]],
                                         ssem0, add=True)
                        gat(r + 1, rows1, sem1).wait()
                        pltpu.async_copy(rows1, acc_sh.at[dst_v.at[r + 1]],
                                         ssem1, add=True)
                        sca(r, rows0, ssem0).wait()
                        gat(lax.rem(r + 2, IB), rows0, sem0).start()
                        sca(r + 1, rows1, ssem1).wait()
                        gat(lax.rem(r + 3, IB), rows1, sem1).start()
                        return carry

                    lax.fori_loop(0, IB // 2, step, 0)
                    gat(0, rows0, sem0).wait()
                    gat(1, rows1, sem1).wait()
                plsc.subcore_barrier()
                pltpu.sync_copy(acc_sh.at[pl.ds(s * SL, SL)],
                                outs[slot].at[pl.ds(s * SL, SL)])
                plsc.subcore_barrier()

    return None


def _sc_aggregate(tables, srcr, dstr, fc, tasks, n_slots):
    zeros = jnp.zeros((SL, fc), jnp.float32)
    body = functools.partial(_agg_body, fc=fc, tasks=tasks,
                             n_tables=len(tables), n_slots=n_slots)
    return pl.kernel(
        body,
        out_type=tuple(jax.ShapeDtypeStruct((NP, fc), jnp.float32)
                       for _ in range(n_slots)),
        mesh=_MESH,
        scratch_types=[
            pltpu.VMEM((IB, 128), jnp.int32),
            pltpu.VMEM((IB, 128), jnp.int32),
            pltpu.VMEM((128, fc), jnp.float32),
            pltpu.VMEM((128, fc), jnp.float32),
            pltpu.VMEM_SHARED((NP, fc), jnp.float32),
            pltpu.SemaphoreType.DMA,
            pltpu.SemaphoreType.DMA,
            pltpu.SemaphoreType.DMA,
            pltpu.SemaphoreType.DMA,
        ],
    )(*tables, srcr, dstr, zeros)


def _norm(deg_blk):
    return lax.rsqrt(jnp.maximum(deg_blk, 1.0))


def _tc1_body(f1_ref, f2_ref, w_ref, dego_ref, *out_refs):
    x = jnp.concatenate([f1_ref[...], f2_ref[...]], axis=1)
    h = jnp.dot(x, w_ref[...], preferred_element_type=jnp.float32)
    h = h * _norm(dego_ref[...])
    for k, o in enumerate(out_refs):
        o[...] = h[:, k * 128:(k + 1) * 128]


def _tc_mid_body(*refs, n_in, n_out, apply_relu):
    a_refs = refs[:n_in]
    degi_ref, b_ref, w_ref, dego_ref = refs[n_in:n_in + 4]
    out_refs = refs[n_in + 4:]
    a = jnp.concatenate([r[...] for r in a_refs], axis=1)
    z = a * _norm(degi_ref[...]) + b_ref[...]
    if apply_relu:
        z = jnp.maximum(z, 0.0)
    h = jnp.dot(z, w_ref[...], preferred_element_type=jnp.float32)
    h = h * _norm(dego_ref[...])
    want = sum(o.shape[1] for o in out_refs)
    if h.shape[1] < want:
        h = jnp.concatenate(
            [h, jnp.zeros((h.shape[0], want - h.shape[1]), h.dtype)], axis=1)
    fc = want // n_out
    for k, o in enumerate(out_refs):
        o[...] = h[:, k * fc:(k + 1) * fc]


def _tc4_body(p0_ref, p1_ref, degi_ref, b_ref, o_ref):
    i = pl.program_id(0)
    p = (p0_ref[...] + p1_ref[...])[:, :64]
    v = p * _norm(degi_ref[...]) + b_ref[...]
    rid = i * RB + lax.broadcasted_iota(jnp.int32, (RB, 64), 0)
    v = jnp.where(rid < N, v, NEG_INF)
    m = jnp.max(v).reshape(1, 1)

    @pl.when(i == 0)
    def _():
        o_ref[...] = m

    @pl.when(i > 0)
    def _():
        o_ref[...] = jnp.maximum(o_ref[...], m)


def _row_spec(width):
    return pl.BlockSpec((RB, width), lambda i: (i, 0))


def _full_spec(shape):
    return pl.BlockSpec(shape, lambda i: tuple(0 for _ in shape))


def _tc1(f1p, f2p, W1, dego):
    return pl.pallas_call(
        _tc1_body,
        grid=(NBLK,),
        in_specs=[_row_spec(256), _row_spec(256), _full_spec((512, 512)),
                  _row_spec(1)],
        out_specs=tuple(_row_spec(128) for _ in range(4)),
        out_shape=tuple(jax.ShapeDtypeStruct((NP, 128), jnp.float32)
                        for _ in range(4)),
    )(f1p, f2p, W1, dego)


def _tc_mid(a_chunks, degi, b, W, dego, n_out, fc_out):
    n_in = len(a_chunks)
    d_in = 128 * n_in
    body = functools.partial(_tc_mid_body, n_in=n_in, n_out=n_out,
                             apply_relu=True)
    return pl.pallas_call(
        body,
        grid=(NBLK,),
        in_specs=[_row_spec(128)] * n_in
        + [_row_spec(1), _full_spec((1, d_in)),
           _full_spec(W.shape), _row_spec(1)],
        out_specs=tuple(_row_spec(fc_out) for _ in range(n_out)),
        out_shape=tuple(jax.ShapeDtypeStruct((NP, fc_out), jnp.float32)
                        for _ in range(n_out)),
    )(*a_chunks, degi, b, W, dego)


def _tc4(p0, p1, degi, b3):
    return pl.pallas_call(
        _tc4_body,
        grid=(NBLK,),
        in_specs=[_row_spec(128), _row_spec(128), _row_spec(1),
                  _full_spec((1, 64))],
        out_specs=pl.BlockSpec((1, 1), lambda i: (0, 0)),
        out_shape=jax.ShapeDtypeStruct((1, 1), jnp.float32),
    )(p0, p1, degi, b3)


def kernel(edge_index, in_feat1, in_feat2, W1, b1, W2, b2, W3, b3):
    src = edge_index[0].astype(jnp.int32)
    dst = edge_index[1].astype(jnp.int32)
    pad = jnp.full((EP - E,), N, jnp.int32)
    srcf = jnp.concatenate([src, pad])
    dstf = jnp.concatenate([dst, pad])
    srcr = srcf.reshape(EROWS, 128)
    dstr = dstf.reshape(EROWS, 128)
    f1p = jnp.pad(in_feat1, ((0, NP - N), (0, 0)))
    f2p = jnp.pad(in_feat2, ((0, NP - N), (0, 0)))

    dego, degi = _sc_degrees(srcf, dstf)
    dego = dego.reshape(NP, 1)
    degi = degi.reshape(NP, 1)

    h1 = _tc1(f1p, f2p, W1, dego)
    all_tasks1 = {0: [(0, 0, RPT, 0), (2, 0, RPT, 2)],
                  1: [(1, 0, RPT, 1), (3, 0, RPT, 3)]}
    a1 = _sc_aggregate(h1, srcr, dstr, 128, all_tasks1, 4)

    h2 = _tc_mid(a1, degi, b1.reshape(1, -1), W2, dego, 2, 128)
    tasks2 = {0: [(0, 0, RPT, 0)], 1: [(1, 0, RPT, 1)]}
    a2 = _sc_aggregate(h2, srcr, dstr, 128, tasks2, 2)

    h3 = _tc_mid(a2, degi, b2.reshape(1, -1), W3, dego, 1, 128)
    tasks3 = {0: [(0, 0, RPT // 2, 0)], 1: [(0, EROWS // 2, RPT // 2, 1)]}
    p0, p1 = _sc_aggregate(h3, srcr, dstr, 128, tasks3, 2)

    out = _tc4(p0, p1, degi, b3.reshape(1, -1))
    return out.reshape(())


# speedup vs baseline: 1.0727x; 1.0727x over previous
import functools

import jax
import jax.numpy as jnp
from jax import lax
from jax.experimental import pallas as pl
from jax.experimental.pallas import tpu as pltpu
from jax.experimental.pallas import tpu_sc as plsc

N = 10000
NP = 10240
RB = 256
NBLK = NP // RB
E = 160000
EROWS = 1280
EP = EROWS * 128
RPT = EROWS // 16
SL = NP // 16
IB = 40

_MESH = plsc.VectorSubcoreMesh(core_axis_name="c", subcore_axis_name="s")
NEG_INF = float("-inf")


EPT = EP // 16


def _deg_body(srcf, dstf, zeros_h, out0, out1,
              idx_v, acc_v, red_v, res_v, stage_sh):
    c = lax.axis_index("c")
    s = lax.axis_index("s")
    ones16 = jnp.ones((16,), jnp.float32)
    pltpu.sync_copy(zeros_h, acc_v)
    for core, eref, out in ((0, srcf, out0), (1, dstf, out1)):
        @pl.when(c == core)
        def _(eref=eref, out=out):
            pltpu.sync_copy(eref.at[pl.ds(s * EPT, EPT)], idx_v)

            def step(b, carry):
                for k in range(8):
                    idx = idx_v[pl.ds(b * 128 + k * 16, 16)]
                    plsc.addupdate_scatter(acc_v, [idx], ones16)
                return carry

            lax.fori_loop(0, EPT // 128, step, 0)
            pltpu.sync_copy(acc_v, stage_sh.at[s])
            plsc.subcore_barrier()
            for t in range(16):
                pltpu.sync_copy(stage_sh.at[t].at[pl.ds(s * SL, SL)],
                                red_v.at[pl.ds(t * SL, SL)])

            def red(i, carry):
                tot = red_v[pl.ds(i * 16, 16)]
                for t in range(1, 16):
                    tot = tot + red_v[pl.ds(t * SL + i * 16, 16)]
                res_v[pl.ds(i * 16, 16)] = tot
                return carry

            lax.fori_loop(0, SL // 16, red, 0)
            pltpu.sync_copy(res_v, out.at[pl.ds(s * SL, SL)])

    return None


def _sc_degrees(srcf, dstf):
    zeros = jnp.zeros((NP,), jnp.float32)
    return pl.kernel(
        _deg_body,
        out_type=(jax.ShapeDtypeStruct((NP,), jnp.float32),
                  jax.ShapeDtypeStruct((NP,), jnp.float32)),
        mesh=_MESH,
        scratch_types=[
            pltpu.VMEM((EPT,), jnp.int32),
            pltpu.VMEM((NP,), jnp.float32),
            pltpu.VMEM((16 * SL,), jnp.float32),
            pltpu.VMEM((SL,), jnp.float32),
            pltpu.VMEM_SHARED((16, NP), jnp.float32),
        ],
        compiler_params=pltpu.CompilerParams(needs_layout_passes=False),
    )(srcf, dstf, zeros)


def _agg_body(*refs, fc, tasks, n_tables, n_slots):
    tables = refs[:n_tables]
    srcr, dstr, zeros_h = refs[n_tables:n_tables + 3]
    outs = refs[n_tables + 3:n_tables + 3 + n_slots]
    src_v, dst_v, rows0, rows1, acc_sh, sem0, sem1 = \
        refs[n_tables + 3 + n_slots:]
    c = lax.axis_index("c")
    s = lax.axis_index("s")
    for core in (0, 1):
        @pl.when(c == core)
        def _(core=core):
            for ci, row_lo, nrt, slot in tasks[core]:
                def gat(row, buf, sem, ci=ci):
                    return pltpu.make_async_copy(
                        tables[ci].at[src_v.at[row]], buf, sem)

                pltpu.sync_copy(zeros_h, acc_sh.at[pl.ds(s * SL, SL)])
                plsc.subcore_barrier()
                for g in range(nrt // IB):
                    base = row_lo + s * nrt + g * IB
                    pltpu.sync_copy(srcr.at[pl.ds(base, IB)], src_v)
                    pltpu.sync_copy(dstr.at[pl.ds(base, IB)], dst_v)
                    gat(0, rows0, sem0).start()
                    gat(1, rows1, sem1).start()

                    def step(h, carry):
                        r = 2 * h
                        gat(r, rows0, sem0).wait()
                        pltpu.sync_copy(rows0, acc_sh.at[dst_v.at[r]],
                                        add=True)
                        gat(lax.rem(r + 2, IB), rows0, sem0).start()
                        gat(r + 1, rows1, sem1).wait()
                        pltpu.sync_copy(rows1, acc_sh.at[dst_v.at[r + 1]],
                                        add=True)
                        gat(lax.rem(r + 3, IB), rows1, sem1).start()
                        return carry

                    lax.fori_loop(0, IB // 2, step, 0)
                    gat(0, rows0, sem0).wait()
                    gat(1, rows1, sem1).wait()
                plsc.subcore_barrier()
                pltpu.sync_copy(acc_sh.at[pl.ds(s * SL, SL)],
                                outs[slot].at[pl.ds(s * SL, SL)])
                plsc.subcore_barrier()

    return None


def _sc_aggregate(tables, srcr, dstr, fc, tasks, n_slots):
    zeros = jnp.zeros((SL, fc), jnp.float32)
    body = functools.partial(_agg_body, fc=fc, tasks=tasks,
                             n_tables=len(tables), n_slots=n_slots)
    return pl.kernel(
        body,
        out_type=tuple(jax.ShapeDtypeStruct((NP, fc), jnp.float32)
                       for _ in range(n_slots)),
        mesh=_MESH,
        scratch_types=[
            pltpu.VMEM((IB, 128), jnp.int32),
            pltpu.VMEM((IB, 128), jnp.int32),
            pltpu.VMEM((128, fc), jnp.float32),
            pltpu.VMEM((128, fc), jnp.float32),
            pltpu.VMEM_SHARED((NP, fc), jnp.float32),
            pltpu.SemaphoreType.DMA,
            pltpu.SemaphoreType.DMA,
        ],
    )(*tables, srcr, dstr, zeros)


def _norm(deg_blk):
    return lax.rsqrt(jnp.maximum(deg_blk, 1.0))


def _tc1_body(f1_ref, f2_ref, w_ref, dego_ref, *out_refs):
    x = jnp.concatenate([f1_ref[...], f2_ref[...]], axis=1)
    h = jnp.dot(x, w_ref[...], preferred_element_type=jnp.float32)
    h = h * _norm(dego_ref[...])
    for k, o in enumerate(out_refs):
        o[...] = h[:, k * 128:(k + 1) * 128]


def _tc_mid_body(*refs, n_in, n_out, apply_relu):
    a_refs = refs[:n_in]
    degi_ref, b_ref, w_ref, dego_ref = refs[n_in:n_in + 4]
    out_refs = refs[n_in + 4:]
    a = jnp.concatenate([r[...] for r in a_refs], axis=1)
    z = a * _norm(degi_ref[...]) + b_ref[...]
    if apply_relu:
        z = jnp.maximum(z, 0.0)
    h = jnp.dot(z, w_ref[...], preferred_element_type=jnp.float32)
    h = h * _norm(dego_ref[...])
    want = sum(o.shape[1] for o in out_refs)
    if h.shape[1] < want:
        h = jnp.concatenate(
            [h, jnp.zeros((h.shape[0], want - h.shape[1]), h.dtype)], axis=1)
    fc = want // n_out
    for k, o in enumerate(out_refs):
        o[...] = h[:, k * fc:(k + 1) * fc]


def _tc4_body(p0_ref, p1_ref, degi_ref, b_ref, o_ref):
    i = pl.program_id(0)
    p = (p0_ref[...] + p1_ref[...])[:, :64]
    v = p * _norm(degi_ref[...]) + b_ref[...]
    rid = i * RB + lax.broadcasted_iota(jnp.int32, (RB, 64), 0)
    v = jnp.where(rid < N, v, NEG_INF)
    m = jnp.max(v).reshape(1, 1)

    @pl.when(i == 0)
    def _():
        o_ref[...] = m

    @pl.when(i > 0)
    def _():
        o_ref[...] = jnp.maximum(o_ref[...], m)


def _row_spec(width):
    return pl.BlockSpec((RB, width), lambda i: (i, 0))


def _full_spec(shape):
    return pl.BlockSpec(shape, lambda i: tuple(0 for _ in shape))


def _tc1(f1p, f2p, W1, dego):
    return pl.pallas_call(
        _tc1_body,
        grid=(NBLK,),
        in_specs=[_row_spec(256), _row_spec(256), _full_spec((512, 512)),
                  _row_spec(1)],
        out_specs=tuple(_row_spec(128) for _ in range(4)),
        out_shape=tuple(jax.ShapeDtypeStruct((NP, 128), jnp.float32)
                        for _ in range(4)),
    )(f1p, f2p, W1, dego)


def _tc_mid(a_chunks, degi, b, W, dego, n_out, fc_out):
    n_in = len(a_chunks)
    d_in = 128 * n_in
    body = functools.partial(_tc_mid_body, n_in=n_in, n_out=n_out,
                             apply_relu=True)
    return pl.pallas_call(
        body,
        grid=(NBLK,),
        in_specs=[_row_spec(128)] * n_in
        + [_row_spec(1), _full_spec((1, d_in)),
           _full_spec(W.shape), _row_spec(1)],
        out_specs=tuple(_row_spec(fc_out) for _ in range(n_out)),
        out_shape=tuple(jax.ShapeDtypeStruct((NP, fc_out), jnp.float32)
                        for _ in range(n_out)),
    )(*a_chunks, degi, b, W, dego)


def _tc4(p0, p1, degi, b3):
    return pl.pallas_call(
        _tc4_body,
        grid=(NBLK,),
        in_specs=[_row_spec(128), _row_spec(128), _row_spec(1),
                  _full_spec((1, 64))],
        out_specs=pl.BlockSpec((1, 1), lambda i: (0, 0)),
        out_shape=jax.ShapeDtypeStruct((1, 1), jnp.float32),
    )(p0, p1, degi, b3)


def kernel(edge_index, in_feat1, in_feat2, W1, b1, W2, b2, W3, b3):
    src = edge_index[0].astype(jnp.int32)
    dst = edge_index[1].astype(jnp.int32)
    pad = jnp.full((EP - E,), N, jnp.int32)
    srcf = jnp.concatenate([src, pad])
    dstf = jnp.concatenate([dst, pad])
    srcr = srcf.reshape(EROWS, 128)
    dstr = dstf.reshape(EROWS, 128)
    f1p = jnp.pad(in_feat1, ((0, NP - N), (0, 0)))
    f2p = jnp.pad(in_feat2, ((0, NP - N), (0, 0)))

    dego, degi = _sc_degrees(srcf, dstf)
    dego = dego.reshape(NP, 1)
    degi = degi.reshape(NP, 1)

    h1 = _tc1(f1p, f2p, W1, dego)
    all_tasks1 = {0: [(0, 0, RPT, 0), (2, 0, RPT, 2)],
                  1: [(1, 0, RPT, 1), (3, 0, RPT, 3)]}
    a1 = _sc_aggregate(h1, srcr, dstr, 128, all_tasks1, 4)

    h2 = _tc_mid(a1, degi, b1.reshape(1, -1), W2, dego, 2, 128)
    tasks2 = {0: [(0, 0, RPT, 0)], 1: [(1, 0, RPT, 1)]}
    a2 = _sc_aggregate(h2, srcr, dstr, 128, tasks2, 2)

    h3 = _tc_mid(a2, degi, b2.reshape(1, -1), W3, dego, 1, 128)
    tasks3 = {0: [(0, 0, RPT // 2, 0)], 1: [(0, EROWS // 2, RPT // 2, 1)]}
    p0, p1 = _sc_aggregate(h3, srcr, dstr, 128, tasks3, 2)

    out = _tc4(p0, p1, degi, b3.reshape(1, -1))
    return out.reshape(())
